# baseline (speedup 1.0000x reference)
import functools

import jax
import jax.numpy as jnp
from jax import lax
from jax.experimental import pallas as pl
from jax.experimental.pallas import tpu as pltpu
from jax.experimental.pallas import tpu_sc as plsc

N = 10000
E = 160000
IN_DIM = 256
HID1 = 256
HALF = 128
OUT_DIM = 128
NUM_GRAPHS = 64

NUM_SC = 2
NUM_TILES = 16

CHUNK = 128
EROWS = E // CHUNK
NPAD = 10240
ROWS_PER_TILE = NPAD // NUM_TILES


K3_ROWS = EROWS // NUM_TILES
K3_EXTRA = EROWS - K3_ROWS * NUM_TILES
K3_G = 6
K3_NG = K3_ROWS // K3_G

EROWS_SC = EROWS // NUM_SC
K1_ROWS = EROWS_SC // NUM_TILES
K1_EXTRA = EROWS_SC - K1_ROWS * NUM_TILES

_mesh = plsc.VectorSubcoreMesh(core_axis_name="c", subcore_axis_name="s")


@functools.partial(
    pl.kernel,
    out_type=jax.ShapeDtypeStruct((NUM_SC, NPAD, HALF), jnp.float32),
    mesh=_mesh,
    scratch_types=[
        pltpu.VMEM((K1_ROWS + 1, 1, CHUNK), jnp.int32),
        pltpu.VMEM((CHUNK, HALF), jnp.float32),
        pltpu.VMEM_SHARED((NPAD, HALF), jnp.float32),
        pltpu.SemaphoreType.DMA,
    ],
)
def _sc_degree(dst2_hbm, zeros_hbm, out_hbm, dstI, ones_v, acc, sem):
    cc = lax.axis_index("c")
    ss = lax.axis_index("s")

    def fill(i, _):
        def fill_j(j, _):
            ones_v[i, pl.ds(j * 16, 16)] = jnp.full((16,), 1.0, jnp.float32)
            return 0
        lax.fori_loop(0, HALF // 16, fill_j, 0)
        return 0
    lax.fori_loop(0, CHUNK, fill, 0)

    row0 = ss * ROWS_PER_TILE
    pltpu.sync_copy(zeros_hbm.at[pl.ds(row0, ROWS_PER_TILE)],
                    acc.at[pl.ds(row0, ROWS_PER_TILE)])

    erow0 = cc * EROWS_SC + ss * K1_ROWS
    nck = jnp.where(ss < K1_EXTRA, K1_ROWS + 1, K1_ROWS)
    pltpu.sync_copy(dst2_hbm.at[pl.ds(erow0, K1_ROWS)],
                    dstI.at[pl.ds(0, K1_ROWS)])

    @pl.when(ss < K1_EXTRA)
    def _():
        pltpu.sync_copy(
            dst2_hbm.at[pl.ds(cc * EROWS_SC + NUM_TILES * K1_ROWS + ss, 1)],
            dstI.at[pl.ds(K1_ROWS, 1)])

    plsc.subcore_barrier()

    def body(i, _):
        pltpu.async_copy(ones_v, acc.at[dstI.at[i, 0]], sem, add=True)
        return 0
    lax.fori_loop(0, nck, body, 0)

    def drain(i, _):
        pltpu.make_async_copy(ones_v, acc.at[dstI.at[i, 0]], sem).wait()
        return 0
    lax.fori_loop(0, nck, drain, 0)

    plsc.subcore_barrier()
    pltpu.sync_copy(acc.at[pl.ds(row0, ROWS_PER_TILE)],
                    out_hbm.at[cc, pl.ds(row0, ROWS_PER_TILE)])


@functools.partial(
    pl.kernel,
    out_type=jax.ShapeDtypeStruct((NUM_SC, NPAD, HALF), jnp.float32),
    mesh=_mesh,
    scratch_types=[
        pltpu.VMEM((2, K3_G, 1, CHUNK), jnp.int32),
        pltpu.VMEM((2, K3_G, 1, CHUNK), jnp.int32),
        pltpu.VMEM((CHUNK, HALF), jnp.float32),
        pltpu.VMEM((CHUNK, HALF), jnp.float32),
        pltpu.VMEM_SHARED((NPAD, HALF), jnp.float32),
        pltpu.SemaphoreType.DMA,
        pltpu.SemaphoreType.DMA,
        pltpu.SemaphoreType.DMA,
    ],
)
def _sc_scatter(src2_hbm, dst2_hbm, tlo_hbm, thi_hbm, zeros_hbm, out_hbm,
                srcI, dstI, rows0, rows1, acc, sem0, sem1, semI):
    cc = lax.axis_index("c")
    ss = lax.axis_index("s")

    row0 = ss * ROWS_PER_TILE
    pltpu.sync_copy(zeros_hbm.at[pl.ds(row0, ROWS_PER_TILE)],
                    acc.at[pl.ds(row0, ROWS_PER_TILE)])

    erow0 = ss * K3_ROWS

    def load_group(g, sl, sync):
        if sync:
            pltpu.sync_copy(src2_hbm.at[pl.ds(erow0 + g * K3_G, K3_G)],
                            srcI.at[sl])
            pltpu.sync_copy(dst2_hbm.at[pl.ds(erow0 + g * K3_G, K3_G)],
                            dstI.at[sl])
        else:
            pltpu.async_copy(src2_hbm.at[pl.ds(erow0 + g * K3_G, K3_G)],
                             srcI.at[sl], semI)
            pltpu.async_copy(dst2_hbm.at[pl.ds(erow0 + g * K3_G, K3_G)],
                             dstI.at[sl], semI)

    def wait_group(g, sl):
        pltpu.make_async_copy(src2_hbm.at[pl.ds(erow0 + g * K3_G, K3_G)],
                              srcI.at[sl], semI).wait()
        pltpu.make_async_copy(dst2_hbm.at[pl.ds(erow0 + g * K3_G, K3_G)],
                              dstI.at[sl], semI).wait()

    plsc.subcore_barrier()

    def run(table):
        load_group(0, 0, True)
        load_group(1, 1, False)
        pltpu.async_copy(table.at[srcI.at[0, 0, 0]], rows0, sem0)

        def group(g, _):
            sl = lax.rem(g, 2)

            def pair(j, _):
                a = 2 * j
                pltpu.async_copy(table.at[srcI.at[sl, a + 1, 0]], rows1, sem1)
                pltpu.make_async_copy(table.at[srcI.at[sl, a, 0]],
                                      rows0, sem0).wait()
                pltpu.sync_copy(rows0, acc.at[dstI.at[sl, a, 0]], add=True)

                @pl.when(a + 2 < K3_G)
                def _():
                    pltpu.async_copy(table.at[srcI.at[sl, a + 2, 0]],
                                     rows0, sem0)

                @pl.when((a + 2 >= K3_G) & (g + 1 < K3_NG))
                def _():
                    wait_group(g + 1, 1 - sl)
                    pltpu.async_copy(table.at[srcI.at[1 - sl, 0, 0]],
                                     rows0, sem0)
                pltpu.make_async_copy(table.at[srcI.at[sl, a + 1, 0]],
                                      rows1, sem1).wait()
                pltpu.sync_copy(rows1, acc.at[dstI.at[sl, a + 1, 0]],
                                add=True)
                return 0
            lax.fori_loop(0, K3_G // 2, pair, 0)

            @pl.when(g + 2 < K3_NG)
            def _():
                load_group(g + 2, sl, False)
            return 0
        lax.fori_loop(0, K3_NG, group, 0)

        @pl.when(ss < K3_EXTRA)
        def _():
            er = NUM_TILES * K3_ROWS + ss
            pltpu.sync_copy(src2_hbm.at[pl.ds(er, 1)], srcI.at[0, pl.ds(0, 1)])
            pltpu.sync_copy(dst2_hbm.at[pl.ds(er, 1)], dstI.at[0, pl.ds(0, 1)])
            pltpu.async_copy(table.at[srcI.at[0, 0, 0]], rows0, sem0).wait()
            pltpu.sync_copy(rows0, acc.at[dstI.at[0, 0, 0]], add=True)

    @pl.when(cc == 0)
    def _():
        run(tlo_hbm)

    @pl.when(cc == 1)
    def _():
        run(thi_hbm)

    plsc.subcore_barrier()
    pltpu.sync_copy(acc.at[pl.ds(row0, ROWS_PER_TILE)],
                    out_hbm.at[cc, pl.ds(row0, ROWS_PER_TILE)])


_RB = 1000


def _k2_body(x_ref, w_ref, degp_ref, out_ref):
    xw = jnp.dot(x_ref[...], w_ref[...], preferred_element_type=jnp.float32)
    deg = degp_ref[0, :, 0:1] + degp_ref[1, :, 0:1] + 1.0
    dis = lax.rsqrt(deg)
    out_ref[0] = dis * xw[:, :HALF]
    out_ref[1] = dis * xw[:, HALF:]


def _tc_matmul_prescale(x, w, degp):
    return pl.pallas_call(
        _k2_body,
        grid=(N // _RB,),
        in_specs=[
            pl.BlockSpec((_RB, IN_DIM), lambda i: (i, 0)),
            pl.BlockSpec((IN_DIM, HID1), lambda i: (0, 0)),
            pl.BlockSpec((NUM_SC, _RB, HALF), lambda i: (0, i, 0)),
        ],
        out_specs=pl.BlockSpec((NUM_SC, _RB, HALF), lambda i: (0, i, 0)),
        out_shape=jax.ShapeDtypeStruct((NUM_SC, N, HALF), jnp.float32),
    )(x, w, degp)


def _k4_body(part_ref, xwp_ref, degp_ref, gb_ref, batch_s_ref, batch_v_ref,
             wlin_ref, blin_ref, out_ref, h_buf, stats, pooled):
    p = pl.program_id(0)
    i = pl.program_id(1)
    nb = pl.num_programs(1)

    @pl.when(p == 0)
    def _():
        deg = degp_ref[0, :, 0:1] + degp_ref[1, :, 0:1] + 1.0
        dis = lax.rsqrt(deg)
        h_lo = dis * (part_ref[0] + xwp_ref[0])
        h_hi = dis * (part_ref[1] + xwp_ref[1])
        h = jnp.concatenate([h_lo, h_hi], axis=1)
        h_buf[pl.ds(i * _RB, _RB), :] = h

        @pl.when(i == 0)
        def _():
            stats[...] = jnp.zeros_like(stats)
        stats[0:1, :] += jnp.sum(h, axis=0, keepdims=True)
        stats[1:2, :] += jnp.sum(h * h, axis=0, keepdims=True)

    @pl.when(p == 1)
    def _():
        mean = stats[0:1, :] / float(N)
        var = stats[1:2, :] / float(N) - mean * mean
        scale = gb_ref[0:1, :] * lax.rsqrt(var + 1e-5)
        shift = gb_ref[1:2, :] - mean * scale
        hn = jnp.maximum(h_buf[pl.ds(i * _RB, _RB), :] * scale + shift, 0.0)

        @pl.when(i == 0)
        def _():
            pooled[...] = jnp.zeros_like(pooled)

        g_lo = batch_s_ref[0, 0]
        g_hi = batch_s_ref[_RB - 1, 0]
        bcol = batch_v_ref[...]

        def seg(g, _):
            m = bcol == g
            masked = jnp.where(m, hn, -1e30)
            bm = jnp.max(masked, axis=0, keepdims=True)
            cur = pooled[pl.ds(g, 1), :]
            pooled[pl.ds(g, 1), :] = jnp.maximum(cur, bm)
            return 0
        lax.fori_loop(g_lo, g_hi + 1, seg, 0)

        @pl.when(i == nb - 1)
        def _():
            out_ref[...] = (
                jnp.dot(pooled[...], wlin_ref[...],
                        preferred_element_type=jnp.float32) + blin_ref[...])


def _tc_final(part, xwp, degp, gb, batch, wlin, blin):
    batch2 = batch.reshape(N, 1)
    return pl.pallas_call(
        _k4_body,
        grid=(2, N // _RB),
        in_specs=[
            pl.BlockSpec((NUM_SC, _RB, HALF), lambda p, i: (0, i * (1 - p), 0)),
            pl.BlockSpec((NUM_SC, _RB, HALF), lambda p, i: (0, i * (1 - p), 0)),
            pl.BlockSpec((NUM_SC, _RB, HALF), lambda p, i: (0, i * (1 - p), 0)),
            pl.BlockSpec((2, HID1), lambda p, i: (0, 0)),
            pl.BlockSpec((_RB, 1), lambda p, i: (i, 0),
                         memory_space=pltpu.SMEM),
            pl.BlockSpec((_RB, 1), lambda p, i: (i, 0)),
            pl.BlockSpec((HID1, OUT_DIM), lambda p, i: (0, 0)),
            pl.BlockSpec((1, OUT_DIM), lambda p, i: (0, 0)),
        ],
        out_specs=pl.BlockSpec((NUM_GRAPHS, OUT_DIM), lambda p, i: (0, 0)),
        out_shape=jax.ShapeDtypeStruct((NUM_GRAPHS, OUT_DIM), jnp.float32),
        scratch_shapes=[
            pltpu.VMEM((N, HID1), jnp.float32),
            pltpu.VMEM((8, HID1), jnp.float32),
            pltpu.VMEM((NUM_GRAPHS, HID1), jnp.float32),
        ],
    )(part, xwp, degp, gb, batch2, batch2, wlin, blin)


def kernel(x, edge_index, batch, W_gcn, b_gcn, gamma, beta, W_lin, b_lin):
    src2 = edge_index[0].reshape(EROWS, 1, CHUNK)
    dst2 = edge_index[1].reshape(EROWS, 1, CHUNK)
    zeros128 = jnp.zeros((NPAD, HALF), jnp.float32)

    degp = _sc_degree(dst2, zeros128)
    xwp = _tc_matmul_prescale(x, W_gcn, degp)
    part = _sc_scatter(src2, dst2, xwp[0], xwp[1], zeros128)
    gb = jnp.stack([gamma, beta])
    out = _tc_final(part, xwp, degp, gb, batch, W_lin,
                    b_lin.reshape(1, OUT_DIM))
    return out
